# baseline (speedup 1.0000x reference)
import functools

import jax
import jax.numpy as jnp
from jax import lax
from jax.experimental import pallas as pl
from jax.experimental.pallas import tpu as pltpu
from jax.experimental.pallas import tpu_sc as plsc

E = 64
D = 1024
N = 8192
S = 2 * N
NW = 32
CHUNK = S // NW
BM = 512
NB = S // BM
NITEMS = NB + E

_mesh = plsc.VectorSubcoreMesh(core_axis_name="c", subcore_axis_name="s")


def _wid():
    return lax.axis_index("s") * 2 + lax.axis_index("c")


def _lanes():
    return lax.iota(jnp.int32, 16)


def _gate_body(x_ref, gw_ref, eid_ref, w_ref):
    xb = x_ref[...]
    logits = lax.dot_general(xb, gw_ref[...],
                             (((1,), (1,)), ((), ())),
                             preferred_element_type=jnp.float32)
    bt = logits.shape[0]
    iota = lax.broadcasted_iota(jnp.int32, (bt, E), 1)
    l0 = jnp.max(logits, axis=1, keepdims=True)
    i0 = jnp.min(jnp.where(logits == l0, iota, E), axis=1, keepdims=True)
    neg = jnp.finfo(jnp.float32).min
    l2 = jnp.where(iota == i0, neg, logits)
    l1 = jnp.max(l2, axis=1, keepdims=True)
    i1 = jnp.min(jnp.where(l2 == l1, iota, E), axis=1, keepdims=True)
    p = jnp.exp(l1 - l0)
    w0 = 1.0 / (1.0 + p)
    eid_ref[...] = jnp.concatenate([i0, i1], axis=1)
    w_ref[...] = jnp.concatenate([w0, 1.0 - w0], axis=1)


def _gate(x, gate_W):
    BT = 512
    return pl.pallas_call(
        _gate_body,
        grid=(N // BT,),
        in_specs=[
            pl.BlockSpec((BT, D), lambda i: (i, 0)),
            pl.BlockSpec((E, D), lambda i: (0, 0)),
        ],
        out_specs=[
            pl.BlockSpec((BT, 2), lambda i: (i, 0)),
            pl.BlockSpec((BT, 2), lambda i: (i, 0)),
        ],
        out_shape=[
            jax.ShapeDtypeStruct((N, 2), jnp.int32),
            jax.ShapeDtypeStruct((N, 2), jnp.float32),
        ],
    )(x, gate_W)


@functools.partial(
    pl.kernel,
    mesh=_mesh,
    compiler_params=pltpu.CompilerParams(needs_layout_passes=False),
    out_type=[
        jax.ShapeDtypeStruct((NW * E * 16,), jnp.int32),
        jax.ShapeDtypeStruct((S,), jnp.int32),
    ],
    scratch_types=[
        pltpu.VMEM((CHUNK,), jnp.int32),
        pltpu.VMEM((E * 16,), jnp.int32),
        pltpu.VMEM((E * 16,), jnp.int32),
        pltpu.VMEM((CHUNK,), jnp.int32),
        pltpu.SemaphoreType.DMA,
    ],
)
def _route1(e_hbm, hist_hbm, rank_hbm, e_v, cnt_v, hc_v, rank_v, sem):
    w = _wid()
    base = w * CHUNK
    pltpu.sync_copy(e_hbm.at[pl.ds(base, CHUNK)], e_v)
    zeros = jnp.zeros((16,), jnp.int32)

    def _zero(e, _):
        cnt_v[pl.ds(e * 16, 16)] = zeros
        return 0
    lax.fori_loop(0, E, _zero, 0)

    lanes = _lanes()

    def _count(k, _):
        ev = e_v[pl.ds(k * 16, 16)]
        idx = ev * 16 + lanes
        c = plsc.load_gather(cnt_v, [idx])
        rank_v[pl.ds(k * 16, 16)] = c
        plsc.store_scatter(cnt_v, [idx], c + 1)
        return 0
    lax.fori_loop(0, CHUNK // 16, _count, 0)

    def _csum(e, _):
        hc_v[pl.ds(e * 16, 16)] = plsc.cumsum(cnt_v[pl.ds(e * 16, 16)])
        return 0
    lax.fori_loop(0, E, _csum, 0)

    pltpu.sync_copy(hc_v, hist_hbm.at[pl.ds(w * (E * 16), E * 16)])
    pltpu.sync_copy(rank_v, rank_hbm.at[pl.ds(base, CHUNK)])


@functools.partial(
    pl.kernel,
    mesh=_mesh,
    compiler_params=pltpu.CompilerParams(needs_layout_passes=False),
    out_type=[
        jax.ShapeDtypeStruct((S,), jnp.int32),
        jax.ShapeDtypeStruct((S,), jnp.int32),
        jax.ShapeDtypeStruct((S,), jnp.float32),
        jax.ShapeDtypeStruct((NITEMS,), jnp.int32),
        jax.ShapeDtypeStruct((NITEMS,), jnp.int32),
        jax.ShapeDtypeStruct((NITEMS,), jnp.int32),
        jax.ShapeDtypeStruct((NITEMS,), jnp.int32),
    ],
    scratch_types=[
        pltpu.VMEM((NW * E * 16,), jnp.int32),
        pltpu.VMEM((CHUNK,), jnp.int32),
        pltpu.VMEM((CHUNK,), jnp.int32),
        pltpu.VMEM((CHUNK,), jnp.float32),
        pltpu.VMEM((E * 16,), jnp.int32),
        pltpu.VMEM((80,), jnp.int32),
        pltpu.VMEM((E,), jnp.int32),
        pltpu.VMEM((CHUNK,), jnp.int32),
        pltpu.VMEM((CHUNK // 128, 128), jnp.int32),
        pltpu.VMEM((CHUNK // 128, 128), jnp.int32),
        pltpu.VMEM((CHUNK // 128, 128), jnp.float32),
        pltpu.VMEM((NITEMS,), jnp.int32),
        pltpu.VMEM((NITEMS,), jnp.int32),
        pltpu.VMEM((NITEMS,), jnp.int32),
        pltpu.VMEM((NITEMS,), jnp.int32),
        pltpu.SemaphoreType.DMA,
    ],
)
def _route2(e_hbm, w_hbm, hist_hbm, rank_hbm,
            pos_hbm, tokS_hbm, wS_hbm, wib_hbm, wie_hbm, wis_hbm, wit_hbm,
            hist_v, e_v, rank_v, w_v, lex_v, off_v, base_v, pos_v,
            pos2_v, tok2_v, w2_v, wib_v, wie_v, wis_v, wit_v, sem):
    w = _wid()
    base = w * CHUNK
    pltpu.sync_copy(e_hbm.at[pl.ds(base, CHUNK)], e_v)
    pltpu.sync_copy(w_hbm.at[pl.ds(base, CHUNK)], w_v)
    pltpu.sync_copy(rank_hbm.at[pl.ds(base, CHUNK)], rank_v)
    pltpu.sync_copy(hist_hbm, hist_v)

    lanes = _lanes()
    lane0 = lanes == 0

    carry = jnp.int32(0)
    for eb in range(E // 16):
        ev16 = (jnp.full((16,), eb * 16, jnp.int32) + lanes) * 16

        def _acc(t, c):
            tot, pref = c
            v = plsc.load_gather(hist_v, [ev16 + (t * E * 16 + 15)])
            tot = tot + v
            pref = pref + jnp.where(t < w, v, 0)
            return (tot, pref)
        tot, pref = lax.fori_loop(
            0, NW, _acc,
            (jnp.zeros((16,), jnp.int32), jnp.zeros((16,), jnp.int32)))
        inc = plsc.cumsum(tot)
        ebase = inc - tot + carry
        off_v[pl.ds(eb * 16, 16)] = ebase
        base_v[pl.ds(eb * 16, 16)] = ebase + pref
        carry = carry + inc[15]
    plsc.store_scatter(off_v, [jnp.full((16,), 64, jnp.int32)],
                       jnp.full((16,), S, jnp.int32), mask=lane0)

    hbase = w * (E * 16)

    def _lex(e, _):
        idx = jnp.full((16,), 0, jnp.int32) + (hbase + e * 16)
        prev = plsc.load_gather(hist_v, [idx + jnp.maximum(lanes - 1, 0)])
        lex_v[pl.ds(e * 16, 16)] = jnp.where(lane0, 0, prev)
        return 0
    lax.fori_loop(0, E, _lex, 0)

    def _pos(k, _):
        ev = e_v[pl.ds(k * 16, 16)]
        b1 = plsc.load_gather(base_v, [ev])
        b2 = plsc.load_gather(lex_v, [ev * 16 + lanes])
        p = b1 + b2 + rank_v[pl.ds(k * 16, 16)]
        pos_v[pl.ds(k * 16, 16)] = p
        slot = jnp.full((16,), base + k * 16, jnp.int32) + lanes
        r = k // 8
        c = (k % 8) * 16
        pos2_v[r, pl.ds(c, 16)] = p
        tok2_v[r, pl.ds(c, 16)] = lax.shift_right_logical(slot, 1)
        w2_v[r, pl.ds(c, 16)] = w_v[pl.ds(k * 16, 16)]
        return 0
    lax.fori_loop(0, CHUNK // 16, _pos, 0)

    pltpu.sync_copy(pos_v, pos_hbm.at[pl.ds(base, CHUNK)])
    handles = []
    for j in range(CHUNK // 128):
        handles.append(
            pltpu.async_copy(tok2_v.at[j], tokS_hbm.at[pos2_v.at[j]], sem))
        handles.append(
            pltpu.async_copy(w2_v.at[j], wS_hbm.at[pos2_v.at[j]], sem))
    for h in handles:
        h.wait()

    @pl.when(w == 0)
    def _items():
        def _wr(ref, i, val):
            plsc.store_scatter(ref, [jnp.full((16,), i, jnp.int32)],
                               jnp.full((16,), 0, jnp.int32) + val, mask=lane0)

        def _per_e(e, cnt):
            ov = off_v[pl.ds(e, 16)]
            s0 = ov[0]
            t0 = ov[1]
            hi = jnp.where(t0 > s0, (t0 + BM - 1) // BM, s0 // BM)

            def _per_b(b, c):
                _wr(wib_v, c, b)
                _wr(wie_v, c, e)
                _wr(wis_v, c, jnp.maximum(s0, b * BM))
                _wr(wit_v, c, jnp.minimum(t0, (b + 1) * BM))
                return c + 1
            return lax.fori_loop(s0 // BM, hi, _per_b, cnt)
        cnt = lax.fori_loop(0, E, _per_e, jnp.int32(0))

        def _pad(c, _):
            _wr(wib_v, c, NB - 1)
            _wr(wie_v, c, E - 1)
            _wr(wis_v, c, 0)
            _wr(wit_v, c, 0)
            return 0
        lax.fori_loop(cnt, NITEMS, _pad, 0)
        pltpu.sync_copy(wib_v, wib_hbm)
        pltpu.sync_copy(wie_v, wie_hbm)
        pltpu.sync_copy(wis_v, wis_hbm)
        pltpu.sync_copy(wit_v, wit_hbm)


GCH = 32
GNC = (S // NW) // GCH

@functools.partial(
    pl.kernel,
    mesh=_mesh,
    compiler_params=pltpu.CompilerParams(needs_layout_passes=False),
    out_type=jax.ShapeDtypeStruct((S, D), jnp.float32),
    scratch_types=[
        pltpu.VMEM((S // NW,), jnp.int32),
        pltpu.VMEM((2, GCH, D), jnp.float32),
        pltpu.SemaphoreType.DMA,
        pltpu.SemaphoreType.DMA,
        pltpu.SemaphoreType.DMA,
        pltpu.SemaphoreType.DMA,
    ],
)
def _gather(x_hbm, tokS_hbm, xs_hbm, idx_v, rows_v, gs0, gs1, ws0, ws1):
    w = _wid()
    base = w * (S // NW)
    pltpu.sync_copy(tokS_hbm.at[pl.ds(base, S // NW)], idx_v)
    gsem = (gs0, gs1)
    wsem = (ws0, ws1)
    gh = [None, None]
    wh = [None, None]
    for g in range(GNC):
        b = g % 2
        if g >= 2:
            wh[b].wait()
        rb = base + g * GCH
        gh[b] = pltpu.async_copy(
            x_hbm.at[idx_v.at[pl.ds(g * GCH, GCH)]], rows_v.at[b], gsem[b])
        if g >= 1:
            pb = 1 - b
            gh[pb].wait()
            wh[pb] = pltpu.async_copy(
                rows_v.at[pb], xs_hbm.at[pl.ds(base + (g - 1) * GCH, GCH)],
                wsem[pb])
    lb = (GNC - 1) % 2
    gh[lb].wait()
    pltpu.sync_copy(rows_v.at[lb], xs_hbm.at[pl.ds(base + (GNC - 1) * GCH, GCH)])
    wh[1 - lb].wait()


KS = 2

def _gmm_body(wib_ref, wie_ref, wis_ref, wit_ref,
              x_ref, ws_ref, W_ref, o_ref):
    i = pl.program_id(0)
    k = pl.program_id(1)
    b = wib_ref[i]

    @pl.when(jnp.logical_and(
        k == 0,
        jnp.logical_or(i == 0, b != wib_ref[jnp.maximum(i - 1, 0)])))
    def _init():
        o_ref[...] = jnp.zeros_like(o_ref)

    y = lax.dot_general(x_ref[...], W_ref[0],
                        (((1,), (1,)), ((), ())),
                        preferred_element_type=jnp.float32)
    y = y * ws_ref[...]
    row = b * BM + lax.broadcasted_iota(jnp.int32, (BM, 1), 0)
    m = jnp.logical_and(row >= wis_ref[i], row < wit_ref[i])
    o_ref[...] += jnp.where(m, y, 0.0)


def _gmm(xs, wS, expert_W, wi_b, wi_e, wi_s, wi_t):
    grid_spec = pltpu.PrefetchScalarGridSpec(
        num_scalar_prefetch=4,
        grid=(NITEMS, KS),
        in_specs=[
            pl.BlockSpec((BM, D // KS), lambda i, k, wb, we, ws, wt: (wb[i], k)),
            pl.BlockSpec((BM, 1), lambda i, k, wb, we, ws, wt: (wb[i], 0)),
            pl.BlockSpec((1, D, D // KS),
                         lambda i, k, wb, we, ws, wt: (we[i], 0, k)),
        ],
        out_specs=pl.BlockSpec((BM, D),
                               lambda i, k, wb, we, ws, wt: (wb[i], 0)),
    )
    return pl.pallas_call(
        _gmm_body,
        grid_spec=grid_spec,
        out_shape=jax.ShapeDtypeStruct((S, D), jnp.float32),
        compiler_params=pltpu.CompilerParams(
            dimension_semantics=("arbitrary", "arbitrary")),
    )(wi_b, wi_e, wi_s, wi_t, xs, wS.reshape(S, 1), expert_W)


CCH = 16
CNC = (N // NW) // CCH

@functools.partial(
    pl.kernel,
    mesh=_mesh,
    compiler_params=pltpu.CompilerParams(needs_layout_passes=False),
    out_type=jax.ShapeDtypeStruct((N, D), jnp.float32),
    scratch_types=[
        pltpu.VMEM((2 * (N // NW),), jnp.int32),
        pltpu.VMEM((2, 2 * CCH, D), jnp.float32),
        pltpu.VMEM((2, CCH, D), jnp.float32),
        pltpu.SemaphoreType.DMA,
        pltpu.SemaphoreType.DMA,
        pltpu.SemaphoreType.DMA,
        pltpu.SemaphoreType.DMA,
    ],
)
def _combine(y_hbm, pos_hbm, out_hbm, idx_v, rows_v, out_v, gs0, gs1, ws0, ws1):
    w = _wid()
    tbase = w * (N // NW)
    pltpu.sync_copy(pos_hbm.at[pl.ds(2 * tbase, 2 * (N // NW))], idx_v)
    gsem = (gs0, gs1)
    wsem = (ws0, ws1)
    gh = [None, None]
    wh = [None, None]

    def _compute(pb):
        def _tok(i, _):
            for d in range(D // 16):
                sl = pl.ds(d * 16, 16)
                out_v[pb, i, sl] = (rows_v[pb, 2 * i, sl]
                                    + rows_v[pb, 2 * i + 1, sl])
            return 0
        lax.fori_loop(0, CCH, _tok, 0)

    for g in range(CNC):
        b = g % 2
        if g >= 2:
            wh[b].wait()
        gh[b] = pltpu.async_copy(
            y_hbm.at[idx_v.at[pl.ds(g * 2 * CCH, 2 * CCH)]], rows_v.at[b],
            gsem[b])
        if g >= 1:
            pb = 1 - b
            gh[pb].wait()
            _compute(pb)
            wh[pb] = pltpu.async_copy(
                out_v.at[pb], out_hbm.at[pl.ds(tbase + (g - 1) * CCH, CCH)],
                wsem[pb])
    lb = (CNC - 1) % 2
    gh[lb].wait()
    _compute(lb)
    pltpu.sync_copy(out_v.at[lb], out_hbm.at[pl.ds(tbase + (CNC - 1) * CCH, CCH)])
    wh[1 - lb].wait()


def kernel(x, gate_W, expert_W):
    eids, wts = _gate(x, gate_W)
    e_flat = eids.reshape(-1)
    w_flat = wts.reshape(-1)
    hist, rank = _route1(e_flat)
    pos, tokS, wS, wi_b, wi_e, wi_s, wi_t = _route2(e_flat, w_flat, hist, rank)
    xs = _gather(x, tokS)
    y = _gmm(xs, wS, expert_W, wi_b, wi_e, wi_s, wi_t)
    return _combine(y, pos)


# speedup vs baseline: 1.2389x; 1.2389x over previous
import functools

import jax
import jax.numpy as jnp
from jax import lax
from jax.experimental import pallas as pl
from jax.experimental.pallas import tpu as pltpu
from jax.experimental.pallas import tpu_sc as plsc

E = 64
D = 1024
N = 8192
S = 2 * N
NW = 32
CHUNK = S // NW
BM = 512
NB = S // BM
NITEMS = NB + E

_mesh = plsc.VectorSubcoreMesh(core_axis_name="c", subcore_axis_name="s")


def _wid():
    return lax.axis_index("s") * 2 + lax.axis_index("c")


def _lanes():
    return lax.iota(jnp.int32, 16)


def _gate_body(x_ref, gw_ref, eid_ref, w_ref):
    xb = x_ref[...]
    logits = lax.dot_general(xb, gw_ref[...],
                             (((1,), (1,)), ((), ())),
                             preferred_element_type=jnp.float32)
    bt = logits.shape[0]
    iota = lax.broadcasted_iota(jnp.int32, (bt, E), 1)
    l0 = jnp.max(logits, axis=1, keepdims=True)
    i0 = jnp.min(jnp.where(logits == l0, iota, E), axis=1, keepdims=True)
    neg = jnp.finfo(jnp.float32).min
    l2 = jnp.where(iota == i0, neg, logits)
    l1 = jnp.max(l2, axis=1, keepdims=True)
    i1 = jnp.min(jnp.where(l2 == l1, iota, E), axis=1, keepdims=True)
    p = jnp.exp(l1 - l0)
    w0 = 1.0 / (1.0 + p)
    eid_ref[...] = jnp.concatenate([i0, i1], axis=1)
    w_ref[...] = jnp.concatenate([w0, 1.0 - w0], axis=1)


def _gate(x, gate_W):
    BT = 512
    return pl.pallas_call(
        _gate_body,
        grid=(N // BT,),
        in_specs=[
            pl.BlockSpec((BT, D), lambda i: (i, 0)),
            pl.BlockSpec((E, D), lambda i: (0, 0)),
        ],
        out_specs=[
            pl.BlockSpec((BT, 2), lambda i: (i, 0)),
            pl.BlockSpec((BT, 2), lambda i: (i, 0)),
        ],
        out_shape=[
            jax.ShapeDtypeStruct((N, 2), jnp.int32),
            jax.ShapeDtypeStruct((N, 2), jnp.float32),
        ],
    )(x, gate_W)


@functools.partial(
    pl.kernel,
    mesh=_mesh,
    compiler_params=pltpu.CompilerParams(needs_layout_passes=False),
    out_type=[
        jax.ShapeDtypeStruct((NW * E * 16,), jnp.int32),
        jax.ShapeDtypeStruct((S,), jnp.int32),
    ],
    scratch_types=[
        pltpu.VMEM((CHUNK,), jnp.int32),
        pltpu.VMEM((E * 16,), jnp.int32),
        pltpu.VMEM((E * 16,), jnp.int32),
        pltpu.VMEM((CHUNK,), jnp.int32),
        pltpu.SemaphoreType.DMA,
    ],
)
def _route1(e_hbm, hist_hbm, rank_hbm, e_v, cnt_v, hc_v, rank_v, sem):
    w = _wid()
    base = w * CHUNK
    pltpu.sync_copy(e_hbm.at[pl.ds(base, CHUNK)], e_v)
    zeros = jnp.zeros((16,), jnp.int32)

    def _zero(e, _):
        cnt_v[pl.ds(e * 16, 16)] = zeros
        return 0
    lax.fori_loop(0, E, _zero, 0)

    lanes = _lanes()

    def _count(k, _):
        ev = e_v[pl.ds(k * 16, 16)]
        idx = ev * 16 + lanes
        c = plsc.load_gather(cnt_v, [idx])
        rank_v[pl.ds(k * 16, 16)] = c
        plsc.store_scatter(cnt_v, [idx], c + 1)
        return 0
    lax.fori_loop(0, CHUNK // 16, _count, 0)

    def _csum(e, _):
        hc_v[pl.ds(e * 16, 16)] = plsc.cumsum(cnt_v[pl.ds(e * 16, 16)])
        return 0
    lax.fori_loop(0, E, _csum, 0)

    pltpu.sync_copy(hc_v, hist_hbm.at[pl.ds(w * (E * 16), E * 16)])
    pltpu.sync_copy(rank_v, rank_hbm.at[pl.ds(base, CHUNK)])


@functools.partial(
    pl.kernel,
    mesh=_mesh,
    compiler_params=pltpu.CompilerParams(needs_layout_passes=False),
    out_type=[
        jax.ShapeDtypeStruct((S,), jnp.int32),
        jax.ShapeDtypeStruct((S,), jnp.int32),
        jax.ShapeDtypeStruct((S,), jnp.float32),
        jax.ShapeDtypeStruct((NITEMS,), jnp.int32),
        jax.ShapeDtypeStruct((NITEMS,), jnp.int32),
        jax.ShapeDtypeStruct((NITEMS,), jnp.int32),
        jax.ShapeDtypeStruct((NITEMS,), jnp.int32),
    ],
    scratch_types=[
        pltpu.VMEM((NW * E * 16,), jnp.int32),
        pltpu.VMEM((CHUNK,), jnp.int32),
        pltpu.VMEM((CHUNK,), jnp.int32),
        pltpu.VMEM((CHUNK,), jnp.float32),
        pltpu.VMEM((E * 16,), jnp.int32),
        pltpu.VMEM((80,), jnp.int32),
        pltpu.VMEM((E,), jnp.int32),
        pltpu.VMEM((CHUNK,), jnp.int32),
        pltpu.VMEM((CHUNK // 128, 128), jnp.int32),
        pltpu.VMEM((CHUNK // 128, 128), jnp.int32),
        pltpu.VMEM((CHUNK // 128, 128), jnp.float32),
        pltpu.VMEM((NITEMS,), jnp.int32),
        pltpu.VMEM((NITEMS,), jnp.int32),
        pltpu.VMEM((NITEMS,), jnp.int32),
        pltpu.VMEM((NITEMS,), jnp.int32),
        pltpu.SemaphoreType.DMA,
    ],
)
def _route2(e_hbm, w_hbm, hist_hbm, rank_hbm,
            pos_hbm, tokS_hbm, wS_hbm, wib_hbm, wie_hbm, wis_hbm, wit_hbm,
            hist_v, e_v, rank_v, w_v, lex_v, off_v, base_v, pos_v,
            pos2_v, tok2_v, w2_v, wib_v, wie_v, wis_v, wit_v, sem):
    w = _wid()
    base = w * CHUNK
    pltpu.sync_copy(e_hbm.at[pl.ds(base, CHUNK)], e_v)
    pltpu.sync_copy(w_hbm.at[pl.ds(base, CHUNK)], w_v)
    pltpu.sync_copy(rank_hbm.at[pl.ds(base, CHUNK)], rank_v)
    pltpu.sync_copy(hist_hbm, hist_v)

    lanes = _lanes()
    lane0 = lanes == 0

    carry = jnp.int32(0)
    for eb in range(E // 16):
        ev16 = (jnp.full((16,), eb * 16, jnp.int32) + lanes) * 16

        def _acc(t, c):
            tot, pref = c
            v = plsc.load_gather(hist_v, [ev16 + (t * E * 16 + 15)])
            tot = tot + v
            pref = pref + jnp.where(t < w, v, 0)
            return (tot, pref)
        tot, pref = lax.fori_loop(
            0, NW, _acc,
            (jnp.zeros((16,), jnp.int32), jnp.zeros((16,), jnp.int32)))
        inc = plsc.cumsum(tot)
        ebase = inc - tot + carry
        off_v[pl.ds(eb * 16, 16)] = ebase
        base_v[pl.ds(eb * 16, 16)] = ebase + pref
        carry = carry + inc[15]
    plsc.store_scatter(off_v, [jnp.full((16,), 64, jnp.int32)],
                       jnp.full((16,), S, jnp.int32), mask=lane0)

    hbase = w * (E * 16)

    def _lex(e, _):
        idx = jnp.full((16,), 0, jnp.int32) + (hbase + e * 16)
        prev = plsc.load_gather(hist_v, [idx + jnp.maximum(lanes - 1, 0)])
        lex_v[pl.ds(e * 16, 16)] = jnp.where(lane0, 0, prev)
        return 0
    lax.fori_loop(0, E, _lex, 0)

    def _pos(k, _):
        ev = e_v[pl.ds(k * 16, 16)]
        b1 = plsc.load_gather(base_v, [ev])
        b2 = plsc.load_gather(lex_v, [ev * 16 + lanes])
        p = b1 + b2 + rank_v[pl.ds(k * 16, 16)]
        pos_v[pl.ds(k * 16, 16)] = p
        slot = jnp.full((16,), base + k * 16, jnp.int32) + lanes
        r = k // 8
        c = (k % 8) * 16
        pos2_v[r, pl.ds(c, 16)] = p
        tok2_v[r, pl.ds(c, 16)] = lax.shift_right_logical(slot, 1)
        w2_v[r, pl.ds(c, 16)] = w_v[pl.ds(k * 16, 16)]
        return 0
    lax.fori_loop(0, CHUNK // 16, _pos, 0)

    pltpu.sync_copy(pos_v, pos_hbm.at[pl.ds(base, CHUNK)])
    handles = []
    for j in range(CHUNK // 128):
        handles.append(
            pltpu.async_copy(tok2_v.at[j], tokS_hbm.at[pos2_v.at[j]], sem))
        handles.append(
            pltpu.async_copy(w2_v.at[j], wS_hbm.at[pos2_v.at[j]], sem))
    for h in handles:
        h.wait()

    @pl.when(w == 0)
    def _items():
        def _wr(ref, i, val):
            plsc.store_scatter(ref, [jnp.full((16,), i, jnp.int32)],
                               jnp.full((16,), 0, jnp.int32) + val, mask=lane0)

        def _per_e(e, cnt):
            ov = off_v[pl.ds(e, 16)]
            s0 = ov[0]
            t0 = ov[1]
            hi = jnp.where(t0 > s0, (t0 + BM - 1) // BM, s0 // BM)

            def _per_b(b, c):
                _wr(wib_v, c, b)
                _wr(wie_v, c, e)
                _wr(wis_v, c, jnp.maximum(s0, b * BM))
                _wr(wit_v, c, jnp.minimum(t0, (b + 1) * BM))
                return c + 1
            return lax.fori_loop(s0 // BM, hi, _per_b, cnt)
        cnt = lax.fori_loop(0, E, _per_e, jnp.int32(0))

        def _pad(c, _):
            _wr(wib_v, c, NB - 1)
            _wr(wie_v, c, E - 1)
            _wr(wis_v, c, 0)
            _wr(wit_v, c, 0)
            return 0
        lax.fori_loop(cnt, NITEMS, _pad, 0)
        pltpu.sync_copy(wib_v, wib_hbm)
        pltpu.sync_copy(wie_v, wie_hbm)
        pltpu.sync_copy(wis_v, wis_hbm)
        pltpu.sync_copy(wit_v, wit_hbm)


GCH = 32
GNC = (S // NW) // GCH

@functools.partial(
    pl.kernel,
    mesh=_mesh,
    compiler_params=pltpu.CompilerParams(needs_layout_passes=False),
    out_type=jax.ShapeDtypeStruct((S, D), jnp.float32),
    scratch_types=[
        pltpu.VMEM((S // NW,), jnp.int32),
        pltpu.VMEM((2, GCH, D), jnp.float32),
        pltpu.SemaphoreType.DMA,
        pltpu.SemaphoreType.DMA,
        pltpu.SemaphoreType.DMA,
        pltpu.SemaphoreType.DMA,
    ],
)
def _gather(x_hbm, tokS_hbm, xs_hbm, idx_v, rows_v, gs0, gs1, ws0, ws1):
    w = _wid()
    base = w * (S // NW)
    pltpu.sync_copy(tokS_hbm.at[pl.ds(base, S // NW)], idx_v)
    gsem = (gs0, gs1)
    wsem = (ws0, ws1)
    gh = [None, None]
    wh = [None, None]
    for g in range(GNC):
        b = g % 2
        if g >= 2:
            wh[b].wait()
        rb = base + g * GCH
        gh[b] = pltpu.async_copy(
            x_hbm.at[idx_v.at[pl.ds(g * GCH, GCH)]], rows_v.at[b], gsem[b])
        if g >= 1:
            pb = 1 - b
            gh[pb].wait()
            wh[pb] = pltpu.async_copy(
                rows_v.at[pb], xs_hbm.at[pl.ds(base + (g - 1) * GCH, GCH)],
                wsem[pb])
    lb = (GNC - 1) % 2
    gh[lb].wait()
    pltpu.sync_copy(rows_v.at[lb], xs_hbm.at[pl.ds(base + (GNC - 1) * GCH, GCH)])
    wh[1 - lb].wait()


def _gmm_body(wib_ref, wie_ref, wis_ref, wit_ref,
              x_ref, ws_ref, W_ref, o_ref):
    i = pl.program_id(0)
    b = wib_ref[i]

    @pl.when(jnp.logical_or(i == 0, b != wib_ref[jnp.maximum(i - 1, 0)]))
    def _init():
        o_ref[...] = jnp.zeros_like(o_ref)

    @pl.when(wis_ref[i] < wit_ref[i])
    def _work():
        y = lax.dot_general(x_ref[...], W_ref[0],
                            (((1,), (1,)), ((), ())),
                            preferred_element_type=jnp.float32)
        y = y * ws_ref[...]
        row = b * BM + lax.broadcasted_iota(jnp.int32, (BM, 1), 0)
        m = jnp.logical_and(row >= wis_ref[i], row < wit_ref[i])
        o_ref[...] += jnp.where(m, y, 0.0)


def _gmm(xs, wS, expert_W, wi_b, wi_e, wi_s, wi_t):
    grid_spec = pltpu.PrefetchScalarGridSpec(
        num_scalar_prefetch=4,
        grid=(NITEMS,),
        in_specs=[
            pl.BlockSpec((BM, D), lambda i, wb, we, ws, wt: (wb[i], 0)),
            pl.BlockSpec((BM, 1), lambda i, wb, we, ws, wt: (wb[i], 0)),
            pl.BlockSpec((1, D, D), lambda i, wb, we, ws, wt: (we[i], 0, 0)),
        ],
        out_specs=pl.BlockSpec((BM, D), lambda i, wb, we, ws, wt: (wb[i], 0)),
    )
    return pl.pallas_call(
        _gmm_body,
        grid_spec=grid_spec,
        out_shape=jax.ShapeDtypeStruct((S, D), jnp.float32),
        compiler_params=pltpu.CompilerParams(
            dimension_semantics=("arbitrary",)),
    )(wi_b, wi_e, wi_s, wi_t, xs, wS.reshape(S, 1), expert_W)


CCH = 16
CNC = (N // NW) // CCH

@functools.partial(
    pl.kernel,
    mesh=_mesh,
    compiler_params=pltpu.CompilerParams(needs_layout_passes=False),
    out_type=jax.ShapeDtypeStruct((N, D), jnp.float32),
    scratch_types=[
        pltpu.VMEM((2 * (N // NW),), jnp.int32),
        pltpu.VMEM((2, 2 * CCH, D), jnp.float32),
        pltpu.VMEM((2, CCH, D), jnp.float32),
        pltpu.SemaphoreType.DMA,
        pltpu.SemaphoreType.DMA,
        pltpu.SemaphoreType.DMA,
        pltpu.SemaphoreType.DMA,
    ],
)
def _combine(y_hbm, pos_hbm, out_hbm, idx_v, rows_v, out_v, gs0, gs1, ws0, ws1):
    w = _wid()
    tbase = w * (N // NW)
    pltpu.sync_copy(pos_hbm.at[pl.ds(2 * tbase, 2 * (N // NW))], idx_v)
    gsem = (gs0, gs1)
    wsem = (ws0, ws1)
    gh = [None, None]
    wh = [None, None]

    def _compute(pb):
        def _tok(i, _):
            for d in range(D // 16):
                sl = pl.ds(d * 16, 16)
                out_v[pb, i, sl] = (rows_v[pb, 2 * i, sl]
                                    + rows_v[pb, 2 * i + 1, sl])
            return 0
        lax.fori_loop(0, CCH, _tok, 0)

    for g in range(CNC):
        b = g % 2
        if g >= 2:
            wh[b].wait()
        gh[b] = pltpu.async_copy(
            y_hbm.at[idx_v.at[pl.ds(g * 2 * CCH, 2 * CCH)]], rows_v.at[b],
            gsem[b])
        if g >= 1:
            pb = 1 - b
            gh[pb].wait()
            _compute(pb)
            wh[pb] = pltpu.async_copy(
                out_v.at[pb], out_hbm.at[pl.ds(tbase + (g - 1) * CCH, CCH)],
                wsem[pb])
    lb = (CNC - 1) % 2
    gh[lb].wait()
    _compute(lb)
    pltpu.sync_copy(out_v.at[lb], out_hbm.at[pl.ds(tbase + (CNC - 1) * CCH, CCH)])
    wh[1 - lb].wait()


def kernel(x, gate_W, expert_W):
    eids, wts = _gate(x, gate_W)
    e_flat = eids.reshape(-1)
    w_flat = wts.reshape(-1)
    hist, rank = _route1(e_flat)
    pos, tokS, wS, wi_b, wi_e, wi_s, wi_t = _route2(e_flat, w_flat, hist, rank)
    xs = _gather(x, tokS)
    y = _gmm(xs, wS, expert_W, wi_b, wi_e, wi_s, wi_t)
    return _combine(y, pos)


# speedup vs baseline: 1.2547x; 1.0127x over previous
import functools

import jax
import jax.numpy as jnp
from jax import lax
from jax.experimental import pallas as pl
from jax.experimental.pallas import tpu as pltpu
from jax.experimental.pallas import tpu_sc as plsc

E = 64
D = 1024
N = 8192
S = 2 * N
NW = 32
CHUNK = S // NW
BM = 512
NB = S // BM
NITEMS = NB + E

_mesh = plsc.VectorSubcoreMesh(core_axis_name="c", subcore_axis_name="s")


def _wid():
    return lax.axis_index("s") * 2 + lax.axis_index("c")


def _lanes():
    return lax.iota(jnp.int32, 16)


def _gate_body(x_ref, gw_ref, eid_ref, w_ref):
    xb = x_ref[...]
    logits = lax.dot_general(xb, gw_ref[...],
                             (((1,), (1,)), ((), ())),
                             preferred_element_type=jnp.float32)
    bt = logits.shape[0]
    iota = lax.broadcasted_iota(jnp.int32, (bt, E), 1)
    l0 = jnp.max(logits, axis=1, keepdims=True)
    i0 = jnp.min(jnp.where(logits == l0, iota, E), axis=1, keepdims=True)
    neg = jnp.finfo(jnp.float32).min
    l2 = jnp.where(iota == i0, neg, logits)
    l1 = jnp.max(l2, axis=1, keepdims=True)
    i1 = jnp.min(jnp.where(l2 == l1, iota, E), axis=1, keepdims=True)
    p = jnp.exp(l1 - l0)
    w0 = 1.0 / (1.0 + p)
    eid_ref[...] = jnp.concatenate([i0, i1], axis=1)
    w_ref[...] = jnp.concatenate([w0, 1.0 - w0], axis=1)


def _gate(x, gate_W):
    BT = 1024
    return pl.pallas_call(
        _gate_body,
        grid=(N // BT,),
        in_specs=[
            pl.BlockSpec((BT, D), lambda i: (i, 0)),
            pl.BlockSpec((E, D), lambda i: (0, 0)),
        ],
        out_specs=[
            pl.BlockSpec((BT, 2), lambda i: (i, 0)),
            pl.BlockSpec((BT, 2), lambda i: (i, 0)),
        ],
        out_shape=[
            jax.ShapeDtypeStruct((N, 2), jnp.int32),
            jax.ShapeDtypeStruct((N, 2), jnp.float32),
        ],
    )(x, gate_W)


@functools.partial(
    pl.kernel,
    mesh=_mesh,
    compiler_params=pltpu.CompilerParams(needs_layout_passes=False),
    out_type=[
        jax.ShapeDtypeStruct((NW * E * 16,), jnp.int32),
        jax.ShapeDtypeStruct((S,), jnp.int32),
    ],
    scratch_types=[
        pltpu.VMEM((CHUNK,), jnp.int32),
        pltpu.VMEM((E * 16,), jnp.int32),
        pltpu.VMEM((E * 16,), jnp.int32),
        pltpu.VMEM((CHUNK,), jnp.int32),
        pltpu.SemaphoreType.DMA,
    ],
)
def _route1(e_hbm, hist_hbm, rank_hbm, e_v, cnt_v, hc_v, rank_v, sem):
    w = _wid()
    base = w * CHUNK
    pltpu.sync_copy(e_hbm.at[pl.ds(base, CHUNK)], e_v)
    zeros = jnp.zeros((16,), jnp.int32)

    def _zero(e, _):
        cnt_v[pl.ds(e * 16, 16)] = zeros
        return 0
    lax.fori_loop(0, E, _zero, 0)

    lanes = _lanes()

    def _count(k, _):
        ev = e_v[pl.ds(k * 16, 16)]
        idx = ev * 16 + lanes
        c = plsc.load_gather(cnt_v, [idx])
        rank_v[pl.ds(k * 16, 16)] = c
        plsc.store_scatter(cnt_v, [idx], c + 1)
        return 0
    lax.fori_loop(0, CHUNK // 16, _count, 0)

    def _csum(e, _):
        hc_v[pl.ds(e * 16, 16)] = plsc.cumsum(cnt_v[pl.ds(e * 16, 16)])
        return 0
    lax.fori_loop(0, E, _csum, 0)

    pltpu.sync_copy(hc_v, hist_hbm.at[pl.ds(w * (E * 16), E * 16)])
    pltpu.sync_copy(rank_v, rank_hbm.at[pl.ds(base, CHUNK)])


@functools.partial(
    pl.kernel,
    mesh=_mesh,
    compiler_params=pltpu.CompilerParams(needs_layout_passes=False),
    out_type=[
        jax.ShapeDtypeStruct((S,), jnp.int32),
        jax.ShapeDtypeStruct((S,), jnp.int32),
        jax.ShapeDtypeStruct((S,), jnp.float32),
        jax.ShapeDtypeStruct((NITEMS,), jnp.int32),
        jax.ShapeDtypeStruct((NITEMS,), jnp.int32),
        jax.ShapeDtypeStruct((NITEMS,), jnp.int32),
        jax.ShapeDtypeStruct((NITEMS,), jnp.int32),
    ],
    scratch_types=[
        pltpu.VMEM((NW * E * 16,), jnp.int32),
        pltpu.VMEM((CHUNK,), jnp.int32),
        pltpu.VMEM((CHUNK,), jnp.int32),
        pltpu.VMEM((CHUNK,), jnp.float32),
        pltpu.VMEM((E * 16,), jnp.int32),
        pltpu.VMEM((80,), jnp.int32),
        pltpu.VMEM((E,), jnp.int32),
        pltpu.VMEM((CHUNK,), jnp.int32),
        pltpu.VMEM((CHUNK // 128, 128), jnp.int32),
        pltpu.VMEM((CHUNK // 128, 128), jnp.int32),
        pltpu.VMEM((CHUNK // 128, 128), jnp.float32),
        pltpu.VMEM((NITEMS,), jnp.int32),
        pltpu.VMEM((NITEMS,), jnp.int32),
        pltpu.VMEM((NITEMS,), jnp.int32),
        pltpu.VMEM((NITEMS,), jnp.int32),
        pltpu.SemaphoreType.DMA,
    ],
)
def _route2(e_hbm, w_hbm, hist_hbm, rank_hbm,
            pos_hbm, tokS_hbm, wS_hbm, wib_hbm, wie_hbm, wis_hbm, wit_hbm,
            hist_v, e_v, rank_v, w_v, lex_v, off_v, base_v, pos_v,
            pos2_v, tok2_v, w2_v, wib_v, wie_v, wis_v, wit_v, sem):
    w = _wid()
    base = w * CHUNK
    pltpu.sync_copy(e_hbm.at[pl.ds(base, CHUNK)], e_v)
    pltpu.sync_copy(w_hbm.at[pl.ds(base, CHUNK)], w_v)
    pltpu.sync_copy(rank_hbm.at[pl.ds(base, CHUNK)], rank_v)
    pltpu.sync_copy(hist_hbm, hist_v)

    lanes = _lanes()
    lane0 = lanes == 0

    carry = jnp.int32(0)
    for eb in range(E // 16):
        ev16 = (jnp.full((16,), eb * 16, jnp.int32) + lanes) * 16

        def _acc(t, c):
            tot, pref = c
            v = plsc.load_gather(hist_v, [ev16 + (t * E * 16 + 15)])
            tot = tot + v
            pref = pref + jnp.where(t < w, v, 0)
            return (tot, pref)
        tot, pref = lax.fori_loop(
            0, NW, _acc,
            (jnp.zeros((16,), jnp.int32), jnp.zeros((16,), jnp.int32)))
        inc = plsc.cumsum(tot)
        ebase = inc - tot + carry
        off_v[pl.ds(eb * 16, 16)] = ebase
        base_v[pl.ds(eb * 16, 16)] = ebase + pref
        carry = carry + inc[15]
    plsc.store_scatter(off_v, [jnp.full((16,), 64, jnp.int32)],
                       jnp.full((16,), S, jnp.int32), mask=lane0)

    hbase = w * (E * 16)

    def _lex(e, _):
        idx = jnp.full((16,), 0, jnp.int32) + (hbase + e * 16)
        prev = plsc.load_gather(hist_v, [idx + jnp.maximum(lanes - 1, 0)])
        lex_v[pl.ds(e * 16, 16)] = jnp.where(lane0, 0, prev)
        return 0
    lax.fori_loop(0, E, _lex, 0)

    def _pos(k, _):
        ev = e_v[pl.ds(k * 16, 16)]
        b1 = plsc.load_gather(base_v, [ev])
        b2 = plsc.load_gather(lex_v, [ev * 16 + lanes])
        p = b1 + b2 + rank_v[pl.ds(k * 16, 16)]
        pos_v[pl.ds(k * 16, 16)] = p
        slot = jnp.full((16,), base + k * 16, jnp.int32) + lanes
        r = k // 8
        c = (k % 8) * 16
        pos2_v[r, pl.ds(c, 16)] = p
        tok2_v[r, pl.ds(c, 16)] = lax.shift_right_logical(slot, 1)
        w2_v[r, pl.ds(c, 16)] = w_v[pl.ds(k * 16, 16)]
        return 0
    lax.fori_loop(0, CHUNK // 16, _pos, 0)

    pltpu.sync_copy(pos_v, pos_hbm.at[pl.ds(base, CHUNK)])
    handles = []
    for j in range(CHUNK // 128):
        handles.append(
            pltpu.async_copy(tok2_v.at[j], tokS_hbm.at[pos2_v.at[j]], sem))
        handles.append(
            pltpu.async_copy(w2_v.at[j], wS_hbm.at[pos2_v.at[j]], sem))
    for h in handles:
        h.wait()

    @pl.when(w == 0)
    def _items():
        def _wr(ref, i, val):
            plsc.store_scatter(ref, [jnp.full((16,), i, jnp.int32)],
                               jnp.full((16,), 0, jnp.int32) + val, mask=lane0)

        def _per_e(e, cnt):
            ov = off_v[pl.ds(e, 16)]
            s0 = ov[0]
            t0 = ov[1]
            hi = jnp.where(t0 > s0, (t0 + BM - 1) // BM, s0 // BM)

            def _per_b(b, c):
                _wr(wib_v, c, b)
                _wr(wie_v, c, e)
                _wr(wis_v, c, jnp.maximum(s0, b * BM))
                _wr(wit_v, c, jnp.minimum(t0, (b + 1) * BM))
                return c + 1
            return lax.fori_loop(s0 // BM, hi, _per_b, cnt)
        cnt = lax.fori_loop(0, E, _per_e, jnp.int32(0))

        def _pad(c, _):
            _wr(wib_v, c, NB - 1)
            _wr(wie_v, c, E - 1)
            _wr(wis_v, c, 0)
            _wr(wit_v, c, 0)
            return 0
        lax.fori_loop(cnt, NITEMS, _pad, 0)
        pltpu.sync_copy(wib_v, wib_hbm)
        pltpu.sync_copy(wie_v, wie_hbm)
        pltpu.sync_copy(wis_v, wis_hbm)
        pltpu.sync_copy(wit_v, wit_hbm)


GCH = 32
GNC = (S // NW) // GCH

@functools.partial(
    pl.kernel,
    mesh=_mesh,
    compiler_params=pltpu.CompilerParams(needs_layout_passes=False),
    out_type=jax.ShapeDtypeStruct((S, D), jnp.float32),
    scratch_types=[
        pltpu.VMEM((S // NW,), jnp.int32),
        pltpu.VMEM((2, GCH, D), jnp.float32),
        pltpu.SemaphoreType.DMA,
        pltpu.SemaphoreType.DMA,
        pltpu.SemaphoreType.DMA,
        pltpu.SemaphoreType.DMA,
    ],
)
def _gather(x_hbm, tokS_hbm, xs_hbm, idx_v, rows_v, gs0, gs1, ws0, ws1):
    w = _wid()
    base = w * (S // NW)
    pltpu.sync_copy(tokS_hbm.at[pl.ds(base, S // NW)], idx_v)
    gsem = (gs0, gs1)
    wsem = (ws0, ws1)
    gh = [None, None]
    wh = [None, None]
    for g in range(GNC):
        b = g % 2
        if g >= 2:
            wh[b].wait()
        rb = base + g * GCH
        gh[b] = pltpu.async_copy(
            x_hbm.at[idx_v.at[pl.ds(g * GCH, GCH)]], rows_v.at[b], gsem[b])
        if g >= 1:
            pb = 1 - b
            gh[pb].wait()
            wh[pb] = pltpu.async_copy(
                rows_v.at[pb], xs_hbm.at[pl.ds(base + (g - 1) * GCH, GCH)],
                wsem[pb])
    lb = (GNC - 1) % 2
    gh[lb].wait()
    pltpu.sync_copy(rows_v.at[lb], xs_hbm.at[pl.ds(base + (GNC - 1) * GCH, GCH)])
    wh[1 - lb].wait()


def _gmm_body(wib_ref, wie_ref, wis_ref, wit_ref,
              x_ref, ws_ref, W_ref, o_ref):
    i = pl.program_id(0)
    b = wib_ref[i]

    @pl.when(jnp.logical_or(i == 0, b != wib_ref[jnp.maximum(i - 1, 0)]))
    def _init():
        o_ref[...] = jnp.zeros_like(o_ref)

    @pl.when(wis_ref[i] < wit_ref[i])
    def _work():
        y = lax.dot_general(x_ref[...], W_ref[0],
                            (((1,), (1,)), ((), ())),
                            preferred_element_type=jnp.float32)
        y = y * ws_ref[...]
        row = b * BM + lax.broadcasted_iota(jnp.int32, (BM, 1), 0)
        m = jnp.logical_and(row >= wis_ref[i], row < wit_ref[i])
        o_ref[...] += jnp.where(m, y, 0.0)


def _gmm(xs, wS, expert_W, wi_b, wi_e, wi_s, wi_t):
    grid_spec = pltpu.PrefetchScalarGridSpec(
        num_scalar_prefetch=4,
        grid=(NITEMS,),
        in_specs=[
            pl.BlockSpec((BM, D), lambda i, wb, we, ws, wt: (wb[i], 0)),
            pl.BlockSpec((BM, 1), lambda i, wb, we, ws, wt: (wb[i], 0)),
            pl.BlockSpec((1, D, D), lambda i, wb, we, ws, wt: (we[i], 0, 0)),
        ],
        out_specs=pl.BlockSpec((BM, D), lambda i, wb, we, ws, wt: (wb[i], 0)),
    )
    return pl.pallas_call(
        _gmm_body,
        grid_spec=grid_spec,
        out_shape=jax.ShapeDtypeStruct((S, D), jnp.float32),
        compiler_params=pltpu.CompilerParams(
            dimension_semantics=("arbitrary",)),
    )(wi_b, wi_e, wi_s, wi_t, xs, wS.reshape(S, 1), expert_W)


CCH = 16
CNC = (N // NW) // CCH

@functools.partial(
    pl.kernel,
    mesh=_mesh,
    compiler_params=pltpu.CompilerParams(needs_layout_passes=False),
    out_type=jax.ShapeDtypeStruct((N, D), jnp.float32),
    scratch_types=[
        pltpu.VMEM((2 * (N // NW),), jnp.int32),
        pltpu.VMEM((2, 2 * CCH, D), jnp.float32),
        pltpu.VMEM((2, CCH, D), jnp.float32),
        pltpu.SemaphoreType.DMA,
        pltpu.SemaphoreType.DMA,
        pltpu.SemaphoreType.DMA,
        pltpu.SemaphoreType.DMA,
    ],
)
def _combine(y_hbm, pos_hbm, out_hbm, idx_v, rows_v, out_v, gs0, gs1, ws0, ws1):
    w = _wid()
    tbase = w * (N // NW)
    pltpu.sync_copy(pos_hbm.at[pl.ds(2 * tbase, 2 * (N // NW))], idx_v)
    gsem = (gs0, gs1)
    wsem = (ws0, ws1)
    gh = [None, None]
    wh = [None, None]

    def _compute(pb):
        def _tok(i, _):
            for d in range(D // 16):
                sl = pl.ds(d * 16, 16)
                out_v[pb, i, sl] = (rows_v[pb, 2 * i, sl]
                                    + rows_v[pb, 2 * i + 1, sl])
            return 0
        lax.fori_loop(0, CCH, _tok, 0)

    for g in range(CNC):
        b = g % 2
        if g >= 2:
            wh[b].wait()
        gh[b] = pltpu.async_copy(
            y_hbm.at[idx_v.at[pl.ds(g * 2 * CCH, 2 * CCH)]], rows_v.at[b],
            gsem[b])
        if g >= 1:
            pb = 1 - b
            gh[pb].wait()
            _compute(pb)
            wh[pb] = pltpu.async_copy(
                out_v.at[pb], out_hbm.at[pl.ds(tbase + (g - 1) * CCH, CCH)],
                wsem[pb])
    lb = (CNC - 1) % 2
    gh[lb].wait()
    _compute(lb)
    pltpu.sync_copy(out_v.at[lb], out_hbm.at[pl.ds(tbase + (CNC - 1) * CCH, CCH)])
    wh[1 - lb].wait()


def kernel(x, gate_W, expert_W):
    eids, wts = _gate(x, gate_W)
    e_flat = eids.reshape(-1)
    w_flat = wts.reshape(-1)
    hist, rank = _route1(e_flat)
    pos, tokS, wS, wi_b, wi_e, wi_s, wi_t = _route2(e_flat, w_flat, hist, rank)
    xs = _gather(x, tokS)
    y = _gmm(xs, wS, expert_W, wi_b, wi_e, wi_s, wi_t)
    return _combine(y, pos)
